# Initial kernel scaffold; baseline (speedup 1.0000x reference)
import jax
import jax.numpy as jnp
from jax.experimental import pallas as pl


def kernel(x, W_int, b_int, W_nh, b_nh, att_w_int, att_w_nh, W_dense, b_dense, src_int, dst_int, src_nh, dst_nh):
    raise NotImplementedError("write your pallas kernel here")


# speedup vs baseline: 21.4810x; 21.4810x over previous
import functools

import jax
import jax.numpy as jnp
from jax import lax
from jax.experimental import pallas as pl
from jax.experimental.pallas import tpu as pltpu
from jax.experimental.pallas import tpu_sc as plsc

L = 2048
B = 64
V = 11
F = 16
E = L * 16
BV = B * V
BF = B * F

_NCOL = 4
_CW = BV // _NCOL
_N_TILES = 16
_ROWS_PER_TILE = L // _N_TILES
_EDGES_PER_TILE = E // _N_TILES
_CHUNK = 128
_N_CHUNKS = _EDGES_PER_TILE // _CHUNK


def _sc_body(x0, x1, x2, x3, srcs_hbm, dsts_hbm, zeros_hbm, o0, o1, o2, o3,
             src_v, dst_v, rows_v, acc_sh, sem):
    xs = (x0, x1, x2, x3)
    outs = (o0, o1, o2, o3)
    c = lax.axis_index("c")
    s = lax.axis_index("s")
    stripe = s * _ROWS_PER_TILE
    ebase = c * E + s * _EDGES_PER_TILE

    for p in range(_NCOL):
        pltpu.sync_copy(zeros_hbm, rows_v)
        pltpu.sync_copy(rows_v, acc_sh.at[pl.ds(stripe, _ROWS_PER_TILE)])
        plsc.subcore_barrier()

        def step(j, carry):
            off = ebase + j * _CHUNK
            pltpu.sync_copy(srcs_hbm.at[pl.ds(off, _CHUNK)], src_v)
            pltpu.sync_copy(dsts_hbm.at[pl.ds(off, _CHUNK)], dst_v)
            pltpu.async_copy(xs[p].at[src_v], rows_v, sem).wait()
            pltpu.sync_copy(rows_v, acc_sh.at[dst_v], add=True)
            return carry

        lax.fori_loop(0, _N_CHUNKS, step, 0)
        plsc.subcore_barrier()

        pltpu.sync_copy(acc_sh.at[pl.ds(stripe, _ROWS_PER_TILE)], rows_v)
        pltpu.sync_copy(rows_v, outs[p].at[pl.ds(c * L + stripe, _ROWS_PER_TILE)])


def _sc_segment_sum(xcols, srcs, dsts, zeros):
    mesh = plsc.VectorSubcoreMesh(core_axis_name="c", subcore_axis_name="s")
    out_t = tuple(jax.ShapeDtypeStruct((2 * L, _CW), jnp.float32)
                  for _ in range(_NCOL))
    kern = functools.partial(
        pl.kernel,
        mesh=mesh,
        out_type=out_t,
        scratch_types=[
            pltpu.VMEM((_CHUNK,), jnp.int32),
            pltpu.VMEM((_CHUNK,), jnp.int32),
            pltpu.VMEM((_ROWS_PER_TILE, _CW), jnp.float32),
            pltpu.VMEM_SHARED((L, _CW), jnp.float32),
            pltpu.SemaphoreType.DMA,
        ],
        compiler_params=pltpu.CompilerParams(use_tc_tiling_on_sc=False),
    )(_sc_body)
    return kern(*xcols, srcs, dsts, zeros)


def _tc_body(agg_ref, wbig_ref, btil_ref, wsel_ref, expand_ref, cw_ref, bd_ref,
             out_ref, num_acc, den_acc):
    c = pl.program_id(0)
    i = pl.program_id(1)

    @pl.when(i == 0)
    def _init():
        num_acc[pl.ds(c, 1), :] = jnp.zeros((1, BF), jnp.float32)
        den_acc[pl.ds(c, 1), :] = jnp.zeros((1, BF), jnp.float32)

    agg = agg_ref[...]
    h = jnp.tanh(
        jnp.dot(agg, wbig_ref[0], preferred_element_type=jnp.float32)
        + btil_ref[0]
    )
    t = jnp.dot(h, wsel_ref[0], preferred_element_type=jnp.float32)
    sc = jnp.exp(jnp.tanh(t))
    s_exp = jnp.dot(sc, expand_ref[...], preferred_element_type=jnp.float32)
    num_acc[pl.ds(c, 1), :] += jnp.sum(s_exp * h, axis=0, keepdims=True)
    den_acc[pl.ds(c, 1), :] += jnp.sum(s_exp, axis=0, keepdims=True)

    @pl.when((c == 1) & (i == pl.num_programs(1) - 1))
    def _fini():
        rep = num_acc[...] / den_acc[...]
        cw = cw_ref[...]
        o = (jnp.dot(rep[0:1, :], cw[0], preferred_element_type=jnp.float32)
             + jnp.dot(rep[1:2, :], cw[1], preferred_element_type=jnp.float32)
             + bd_ref[...])
        out_ref[...] = o


def _tc_fused(agg, wbig, btil, wsel, expand, cw, bd, interpret=False):
    n_blk = 8
    rows = (2 * L) // (2 * n_blk)
    return pl.pallas_call(
        _tc_body,
        grid=(2, n_blk),
        in_specs=[
            pl.BlockSpec((rows, BV), lambda c, i: (c * n_blk + i, 0)),
            pl.BlockSpec((1, BV, BF), lambda c, i: (c, 0, 0)),
            pl.BlockSpec((1, 1, BF), lambda c, i: (c, 0, 0)),
            pl.BlockSpec((1, BF, B), lambda c, i: (c, 0, 0)),
            pl.BlockSpec((B, BF), lambda c, i: (0, 0)),
            pl.BlockSpec((2, BF, B), lambda c, i: (0, 0, 0)),
            pl.BlockSpec((1, 1), lambda c, i: (0, 0)),
        ],
        out_specs=pl.BlockSpec((1, B), lambda c, i: (0, 0)),
        out_shape=jax.ShapeDtypeStruct((1, B), jnp.float32),
        scratch_shapes=[
            pltpu.VMEM((2, BF), jnp.float32),
            pltpu.VMEM((2, BF), jnp.float32),
        ],
        interpret=interpret,
    )(agg, wbig, btil, wsel, expand, cw, bd)


def kernel(x, W_int, b_int, W_nh, b_nh, att_w_int, att_w_nh, W_dense, b_dense,
           src_int, dst_int, src_nh, dst_nh):
    x2 = x.reshape(L, BV)
    xcols = tuple(x2[:, p * _CW:(p + 1) * _CW] for p in range(_NCOL))
    srcs = jnp.concatenate([src_int, src_nh]).astype(jnp.int32)
    dsts = jnp.concatenate([dst_int, dst_nh]).astype(jnp.int32)
    zeros = jnp.zeros((_ROWS_PER_TILE, _CW), jnp.float32)

    aggs = _sc_segment_sum(xcols, srcs, dsts, zeros)
    agg = jnp.concatenate(aggs, axis=1)

    eye = jnp.eye(B, dtype=jnp.float32)
    wbig = jnp.stack([jnp.kron(eye, W_int), jnp.kron(eye, W_nh)])
    btil = jnp.stack([jnp.tile(b_int, B), jnp.tile(b_nh, B)]).reshape(2, 1, BF)
    wsel = jnp.stack([jnp.kron(eye, att_w_int.reshape(F, 1)),
                      jnp.kron(eye, att_w_nh.reshape(F, 1))])
    expand = jnp.kron(eye, jnp.ones((1, F), jnp.float32))
    cw = jnp.stack([jnp.kron(eye, W_dense[:F].astype(jnp.float32)),
                    jnp.kron(eye, W_dense[F:].astype(jnp.float32))])
    bd = b_dense.reshape(1, 1)

    out = _tc_fused(agg, wbig, btil, wsel, expand, cw, bd)
    return out.reshape(B)
